# baseline (speedup 1.0000x reference)
import jax
import jax.numpy as jnp
from jax.experimental import pallas as pl

_BN = 8192
_KB = 1024
_LANES = 128
_IMAX = 2**31 - 1


def _norms_body(x_ref, c_ref, x2_ref, c2_ref):
    x = x_ref[...]
    c = c_ref[...]
    x2_ref[...] = jnp.sum(x * x, axis=1, keepdims=True)
    c2_ref[...] = jnp.sum(c * c, axis=1)[None, :]


def _body(x_ref, ct_ref, x2_ref, c2_ref, idx_ref, dist_ref):
    x = x_ref[...]
    x2x = x + x
    x2 = x2_ref[...]
    c2 = c2_ref[...]
    kk = ct_ref.shape[1]

    run_val = None
    for kb in range(kk // _KB):
        cbt = ct_ref[:, kb * _KB:(kb + 1) * _KB]
        dot2 = jax.lax.dot_general(
            x2x, cbt, (((1,), (0,)), ((), ())),
            preferred_element_type=jnp.float32)
        for cj in range(_KB // _LANES):
            ci = kb * (_KB // _LANES) + cj
            sl = slice(cj * _LANES, (cj + 1) * _LANES)
            d2c = (x2 + c2[:, ci * _LANES:(ci + 1) * _LANES]) - dot2[:, sl]
            if run_val is None:
                run_val = d2c
                run_cid = jnp.zeros(d2c.shape, jnp.int32)
            else:
                m = d2c < run_val
                run_val = jnp.where(m, d2c, run_val)
                run_cid = jnp.where(m, jnp.int32(ci), run_cid)

    gidx = (run_cid * _LANES
            + jax.lax.broadcasted_iota(jnp.int32, run_val.shape, 1))
    lmin = jnp.min(run_val, axis=1, keepdims=True)
    idx_ref[...] = jnp.min(
        jnp.where(run_val == lmin, gidx, jnp.int32(_IMAX)),
        axis=1, keepdims=True)
    dist_ref[...] = jnp.sqrt(jnp.maximum(lmin, 1e-12))


def kernel(state, centroids):
    if state.ndim == 1:
        state = state[None, :]
    n, d = state.shape
    kk, _ = centroids.shape

    x2, c2 = pl.pallas_call(
        _norms_body,
        out_shape=[
            jax.ShapeDtypeStruct((n, 1), jnp.float32),
            jax.ShapeDtypeStruct((1, kk), jnp.float32),
        ],
    )(state, centroids)

    ct = centroids.T

    grid = (n // _BN,)
    idx2, dist2 = pl.pallas_call(
        _body,
        grid=grid,
        in_specs=[
            pl.BlockSpec((_BN, d), lambda i: (i, 0)),
            pl.BlockSpec((d, kk), lambda i: (0, 0)),
            pl.BlockSpec((_BN, 1), lambda i: (i, 0)),
            pl.BlockSpec((1, kk), lambda i: (0, 0)),
        ],
        out_specs=[
            pl.BlockSpec((_BN, 1), lambda i: (i, 0)),
            pl.BlockSpec((_BN, 1), lambda i: (i, 0)),
        ],
        out_shape=[
            jax.ShapeDtypeStruct((n, 1), jnp.int32),
            jax.ShapeDtypeStruct((n, 1), jnp.float32),
        ],
    )(state, ct, x2, c2)
    return idx2[:, 0], dist2[:, 0]


# speedup vs baseline: 1.3249x; 1.3249x over previous
import jax
import jax.numpy as jnp
from jax.experimental import pallas as pl

_BN = 4096
_KB = 512
_LANES = 128
_IMAX = 2**31 - 1


def _norms_body(x_ref, c_ref, x2_ref, c2_ref):
    x = x_ref[...]
    c = c_ref[...]
    x2_ref[...] = jnp.sum(x * x, axis=1, keepdims=True)
    c2_ref[...] = jnp.sum(c * c, axis=1)[None, :]


def _body(x_ref, ct_ref, x2_ref, c2_ref, idx_ref, dist_ref):
    x = x_ref[...]
    x2x = x + x
    x2 = x2_ref[...]
    c2 = c2_ref[...]
    kk = ct_ref.shape[1]

    run_val = None
    for kb in range(kk // _KB):
        cbt = ct_ref[:, kb * _KB:(kb + 1) * _KB]
        dot2 = jax.lax.dot_general(
            x2x, cbt, (((1,), (0,)), ((), ())),
            preferred_element_type=jnp.float32)
        for cj in range(_KB // _LANES):
            ci = kb * (_KB // _LANES) + cj
            sl = slice(cj * _LANES, (cj + 1) * _LANES)
            d2c = (x2 + c2[:, ci * _LANES:(ci + 1) * _LANES]) - dot2[:, sl]
            if run_val is None:
                run_val = d2c
                run_cid = jnp.zeros(d2c.shape, jnp.int32)
            else:
                m = d2c < run_val
                run_val = jnp.where(m, d2c, run_val)
                run_cid = jnp.where(m, jnp.int32(ci), run_cid)

    gidx = (run_cid * _LANES
            + jax.lax.broadcasted_iota(jnp.int32, run_val.shape, 1))
    lmin = jnp.min(run_val, axis=1, keepdims=True)
    idx_ref[...] = jnp.min(
        jnp.where(run_val == lmin, gidx, jnp.int32(_IMAX)),
        axis=1, keepdims=True)
    dist_ref[...] = jnp.sqrt(jnp.maximum(lmin, 1e-12))


def kernel(state, centroids):
    if state.ndim == 1:
        state = state[None, :]
    n, d = state.shape
    kk, _ = centroids.shape

    x2, c2 = pl.pallas_call(
        _norms_body,
        out_shape=[
            jax.ShapeDtypeStruct((n, 1), jnp.float32),
            jax.ShapeDtypeStruct((1, kk), jnp.float32),
        ],
    )(state, centroids)

    ct = centroids.T

    grid = (n // _BN,)
    idx2, dist2 = pl.pallas_call(
        _body,
        grid=grid,
        in_specs=[
            pl.BlockSpec((_BN, d), lambda i: (i, 0)),
            pl.BlockSpec((d, kk), lambda i: (0, 0)),
            pl.BlockSpec((_BN, 1), lambda i: (i, 0)),
            pl.BlockSpec((1, kk), lambda i: (0, 0)),
        ],
        out_specs=[
            pl.BlockSpec((_BN, 1), lambda i: (i, 0)),
            pl.BlockSpec((_BN, 1), lambda i: (i, 0)),
        ],
        out_shape=[
            jax.ShapeDtypeStruct((n, 1), jnp.int32),
            jax.ShapeDtypeStruct((n, 1), jnp.float32),
        ],
    )(state, ct, x2, c2)
    return idx2[:, 0], dist2[:, 0]
